# baseline (speedup 1.0000x reference)
import functools

import jax
import jax.numpy as jnp
from jax import lax
from jax.experimental import pallas as pl
from jax.experimental.pallas import tpu as pltpu
from jax.experimental.pallas import tpu_sc as plsc

_B = 128
_L = 16
_PAD = 160

_mesh = plsc.VectorSubcoreMesh(core_axis_name="c", subcore_axis_name="s")


@functools.partial(
    pl.kernel,
    mesh=_mesh,
    out_type=jax.ShapeDtypeStruct((_PAD,), jnp.int32),
    scratch_types=[
        pltpu.VMEM((_B,), jnp.int32),
        pltpu.VMEM((_PAD,), jnp.int32),
    ],
    compiler_params=pltpu.CompilerParams(needs_layout_passes=False),
)
def _row_splits_sc(rl_hbm, out_hbm, rl_v, out_v):
    @pl.when((lax.axis_index("c") == 0) & (lax.axis_index("s") == 0))
    def _():
        pltpu.sync_copy(rl_hbm, rl_v)
        carry = jnp.int32(0)
        for j in range(_B // _L):
            x = rl_v[pl.ds(j * _L, _L)]
            inc = plsc.cumsum(x)
            out_v[pl.ds(j * _L, _L)] = (inc - x) + carry
            carry = carry + jnp.sum(x)
        total = jnp.zeros((_L,), jnp.int32) + carry
        for j in range(_B // _L, _PAD // _L):
            out_v[pl.ds(j * _L, _L)] = total
        pltpu.sync_copy(out_v, out_hbm)


def kernel(values, row_lengths):
    splits_padded = _row_splits_sc(row_lengths)
    row_splits = lax.slice(splits_padded, (0,), (_B + 1,))
    return values, row_splits
